# baseline (speedup 1.0000x reference)
import numpy as np
import jax
import jax.numpy as jnp
from jax import lax
from jax.experimental import pallas as pl
from jax.experimental.pallas import tpu as pltpu, tpu_sc as plsc

_B = 16384
_N_CONT = 13
_N_BIN = 13
_N_CAT = 26
_VOCAB = 100000
_EMB = 16
_EPS = 1e-5

_NC = 2
_NS = 16
_NW = _NC * _NS
_CHUNK = 128
_GROUP = 8
_GROUP_ROWS = _CHUNK * _GROUP


def _sc_detile(tt):
    W = 1024
    nblk = _VOCAB // W
    remw = _VOCAB - nblk * W
    fsz = _VOCAB * _EMB
    bsz = W * _EMB

    mesh = plsc.VectorSubcoreMesh(core_axis_name="c", subcore_axis_name="s")

    def body(tt_hbm, out_hbm, in_v, out_v, in2_v, out2_v, sem, sem_o):
        wid = lax.axis_index("s") * _NC + lax.axis_index("c")
        iot = lax.iota(jnp.int32, 16)
        flat_base = iot * _EMB

        @pl.when(wid < _N_CAT)
        def _():
            f = wid

            def in_copy(b):
                return pltpu.make_async_copy(
                    tt_hbm.at[f, :, pl.ds(b * W, W)],
                    in_v.at[pl.ds((b % 2) * 16, 16)],
                    sem,
                )

            def out_copy(b):
                return pltpu.make_async_copy(
                    out_v.at[pl.ds((b % 2) * bsz, bsz)],
                    out_hbm.at[pl.ds(f * fsz + b * bsz, bsz)],
                    sem_o,
                )

            in_copy(0).start()

            def blk(b, carry):
                @pl.when(b + 1 < nblk)
                def _():
                    in_copy(b + 1).start()

                @pl.when(b >= 2)
                def _():
                    out_copy(b - 2).wait()
                in_copy(b).wait()
                boff = (b % 2) * 16
                ooff = (b % 2) * bsz
                rows = [jnp.full((16,), boff + e, jnp.int32)
                        for e in range(_EMB)]
                for j in range(W // 16):
                    lanes = j * 16 + iot
                    base_j = jnp.full((16,), ooff + j * 256, jnp.int32) \
                        + flat_base
                    for e in range(_EMB):
                        vals = plsc.load_gather(in_v, [rows[e], lanes])
                        plsc.store_scatter(out_v, [base_j + e], vals)
                out_copy(b).start()
                return carry

            lax.fori_loop(0, nblk, blk, 0)
            out_copy(nblk - 2).wait()
            out_copy(nblk - 1).wait()

            pltpu.sync_copy(tt_hbm.at[f, :, pl.ds(nblk * W, remw)], in2_v)
            for j in range(remw // 16):
                for e in range(_EMB):
                    vals = plsc.load_gather(
                        in2_v,
                        [jnp.full((16,), e, jnp.int32), j * 16 + iot])
                    plsc.store_scatter(
                        out2_v, [j * 256 + flat_base + e], vals)
            pltpu.sync_copy(
                out2_v,
                out_hbm.at[pl.ds(f * fsz + nblk * bsz, remw * _EMB)])

    return pl.kernel(
        body,
        out_type=jax.ShapeDtypeStruct((_N_CAT * fsz,), jnp.float32),
        mesh=mesh,
        scratch_types=[
            pltpu.VMEM((32, W), jnp.float32),
            pltpu.VMEM((2 * bsz,), jnp.float32),
            pltpu.VMEM((16, remw), jnp.float32),
            pltpu.VMEM((remw * _EMB,), jnp.float32),
            pltpu.SemaphoreType.DMA,
            pltpu.SemaphoreType.DMA,
        ],
        compiler_params=pltpu.CompilerParams(use_tc_tiling_on_sc=True,
                                             needs_layout_passes=False),
    )(tt)


def _sc_gather(tables_flat, idx2d):
    n_chunks = idx2d.shape[0]
    nrows = n_chunks * _CHUNK
    chunks_per_w = n_chunks // _NW
    groups = chunks_per_w // _GROUP

    mesh = plsc.VectorSubcoreMesh(core_axis_name="c", subcore_axis_name="s")

    def body(tables_hbm, idx_hbm, out_hbm, idx_v, rows_v, sem):
        wid = lax.axis_index("s") * _NC + lax.axis_index("c")
        crow = wid * chunks_per_w
        pltpu.sync_copy(idx_hbm.at[pl.ds(crow, chunks_per_w)], idx_v)

        def group(g, carry):
            descs = []
            for s in range(_GROUP):
                j = g * _GROUP + s
                d = pltpu.async_copy(
                    tables_hbm.at[idx_v.at[j]],
                    rows_v.at[pl.ds(s * _CHUNK, _CHUNK)],
                    sem,
                )
                descs.append(d)
            for d in descs:
                d.wait()
            pltpu.sync_copy(
                rows_v,
                out_hbm.at[pl.ds((crow + g * _GROUP) * _CHUNK, _GROUP_ROWS)],
            )
            return carry

        lax.fori_loop(0, groups, group, 0)

    return pl.kernel(
        body,
        out_type=jax.ShapeDtypeStruct((nrows, _EMB), jnp.float32),
        mesh=mesh,
        scratch_types=[
            pltpu.VMEM((chunks_per_w, _CHUNK), jnp.int32),
            pltpu.VMEM((_GROUP_ROWS, _EMB), jnp.float32),
            pltpu.SemaphoreType.DMA,
        ],
        compiler_params=pltpu.CompilerParams(use_tc_tiling_on_sc=False),
    )(tables_flat, idx2d)


def _tc_mlp(xw, emb, Ww, W1, W2, W3, b1r, g1r, be1r, b2r, g2r, be2r, c0):
    B, deep_emb = emb.shape
    R = 2048
    inv = float(1.0 / np.sqrt(1.0 + _EPS))

    def body(xw_ref, emb_ref, ww_ref, w1_ref, w2_ref, w3_ref,
             b1_ref, g1_ref, be1_ref, b2_ref, g2_ref, be2_ref, c0_ref,
             out_ref):
        xw_blk = xw_ref[...]
        emb_blk = emb_ref[...]
        dn = (((1,), (1,)), ((), ()))
        hi = jax.lax.Precision.HIGHEST
        wide = lax.dot_general(xw_blk, ww_ref[...], dn, precision=hi,
                               preferred_element_type=jnp.float32)
        w1 = w1_ref[...]
        h = lax.dot_general(xw_blk[:, :26], w1[:, :26], dn, precision=hi,
                            preferred_element_type=jnp.float32)
        h = h + lax.dot_general(emb_blk, w1[:, 26:], dn, precision=hi,
                                preferred_element_type=jnp.float32)
        h = (h + b1_ref[...]) * (g1_ref[...] * inv) + be1_ref[...]
        h = jnp.maximum(h, 0.0)
        h = lax.dot_general(h, w2_ref[...], dn, precision=hi,
                            preferred_element_type=jnp.float32)
        h = (h + b2_ref[...]) * (g2_ref[...] * inv) + be2_ref[...]
        h = jnp.maximum(h, 0.0)
        deep = lax.dot_general(h, w3_ref[...], dn, precision=hi,
                               preferred_element_type=jnp.float32)
        out_ref[...] = 0.5 * wide + 0.5 * deep + c0_ref[...]

    full = lambda shape: pl.BlockSpec(shape, lambda i: (0,) * len(shape))
    return pl.pallas_call(
        body,
        grid=(B // R,),
        in_specs=[
            pl.BlockSpec((R, xw.shape[1]), lambda i: (i, 0)),
            pl.BlockSpec((R, deep_emb), lambda i: (i, 0)),
            full(Ww.shape),
            full(W1.shape),
            full(W2.shape),
            full(W3.shape),
            full(b1r.shape),
            full(g1r.shape),
            full(be1r.shape),
            full(b2r.shape),
            full(g2r.shape),
            full(be2r.shape),
            full(c0.shape),
        ],
        out_specs=pl.BlockSpec((R, 1), lambda i: (i, 0)),
        out_shape=jax.ShapeDtypeStruct((B, 1), jnp.float32),
    )(xw, emb, Ww, W1, W2, W3, b1r, g1r, be1r, b2r, g2r, be2r, c0)


def kernel(continuous, binary, categorical, W_wide, b_wide, emb_tables,
           W1, b1, g1, be1, W2, b2, g2, be2, W3, b3):
    B = continuous.shape[0]
    catf = categorical.astype(jnp.float32)
    xw = jnp.concatenate([continuous, binary, catf], axis=1)

    cat32 = categorical.astype(jnp.int32)
    offs = jnp.arange(_N_CAT, dtype=jnp.int32) * _VOCAB
    idx2d = (cat32 + offs[None, :]).reshape(-1, _CHUNK)

    tt = jnp.transpose(emb_tables, (0, 2, 1))
    tables_flat = _sc_detile(tt).reshape(_N_CAT * _VOCAB, _EMB)
    del tt
    gathered = _sc_gather(tables_flat, idx2d)
    emb_flat = gathered.reshape(B, _N_CAT * _EMB)

    c0 = (0.5 * (b_wide + b3)).reshape(1, 1)
    out2d = _tc_mlp(
        xw, emb_flat, W_wide, W1, W2, W3,
        b1.reshape(1, -1), g1.reshape(1, -1), be1.reshape(1, -1),
        b2.reshape(1, -1), g2.reshape(1, -1), be2.reshape(1, -1),
        c0,
    )
    return out2d.reshape(B)


# speedup vs baseline: 1.4549x; 1.4549x over previous
import numpy as np
import jax
import jax.numpy as jnp
from jax import lax
from jax.experimental import pallas as pl
from jax.experimental.pallas import tpu as pltpu, tpu_sc as plsc

_B = 16384
_N_CONT = 13
_N_BIN = 13
_N_CAT = 26
_VOCAB = 100000
_EMB = 16
_EPS = 1e-5

_NC = 2
_NS = 16
_NW = _NC * _NS
_CHUNK = 128
_GROUP = 8
_GROUP_ROWS = _CHUNK * _GROUP


def _sc_detile(tt):
    W = 1024
    nblk = _VOCAB // W
    remw = _VOCAB - nblk * W
    fsz = _VOCAB * _EMB
    bsz = W * _EMB

    mesh = plsc.VectorSubcoreMesh(core_axis_name="c", subcore_axis_name="s")

    def body(tt_hbm, out_hbm, in_v, out_v, in2_v, out2_v, sem, sem_o):
        wid = lax.axis_index("s") * _NC + lax.axis_index("c")
        iot = lax.iota(jnp.int32, 16)
        flat_base = iot * _EMB

        @pl.when(wid < _N_CAT)
        def _():
            f = wid

            def in_copy(b):
                return pltpu.make_async_copy(
                    tt_hbm.at[f, :, pl.ds(b * W, W)],
                    in_v.at[pl.ds((b % 2) * 16, 16)],
                    sem,
                )

            def out_copy(b):
                return pltpu.make_async_copy(
                    out_v.at[pl.ds((b % 2) * bsz, bsz)],
                    out_hbm.at[pl.ds(f * fsz + b * bsz, bsz)],
                    sem_o,
                )

            in_copy(0).start()

            def blk(b, carry):
                @pl.when(b + 1 < nblk)
                def _():
                    in_copy(b + 1).start()

                @pl.when(b >= 2)
                def _():
                    out_copy(b - 2).wait()
                in_copy(b).wait()
                boff = (b % 2) * 16
                ooff = (b % 2) * bsz
                rows = [jnp.full((16,), boff + e, jnp.int32)
                        for e in range(_EMB)]
                for j in range(W // 16):
                    lanes = j * 16 + iot
                    base_j = jnp.full((16,), ooff + j * 256, jnp.int32) \
                        + flat_base
                    vals = [plsc.load_gather(in_v, [rows[e], lanes])
                            for e in range(_EMB)]
                    for e in range(_EMB):
                        plsc.store_scatter(out_v, [base_j + e], vals[e])
                out_copy(b).start()
                return carry

            lax.fori_loop(0, nblk, blk, 0)
            out_copy(nblk - 2).wait()
            out_copy(nblk - 1).wait()

            pltpu.sync_copy(tt_hbm.at[f, :, pl.ds(nblk * W, remw)], in2_v)
            for j in range(remw // 16):
                for e in range(_EMB):
                    vals = plsc.load_gather(
                        in2_v,
                        [jnp.full((16,), e, jnp.int32), j * 16 + iot])
                    plsc.store_scatter(
                        out2_v, [j * 256 + flat_base + e], vals)
            pltpu.sync_copy(
                out2_v,
                out_hbm.at[pl.ds(f * fsz + nblk * bsz, remw * _EMB)])

    return pl.kernel(
        body,
        out_type=jax.ShapeDtypeStruct((_N_CAT * fsz,), jnp.float32),
        mesh=mesh,
        scratch_types=[
            pltpu.VMEM((32, W), jnp.float32),
            pltpu.VMEM((2 * bsz,), jnp.float32),
            pltpu.VMEM((16, remw), jnp.float32),
            pltpu.VMEM((remw * _EMB,), jnp.float32),
            pltpu.SemaphoreType.DMA,
            pltpu.SemaphoreType.DMA,
        ],
        compiler_params=pltpu.CompilerParams(use_tc_tiling_on_sc=True,
                                             needs_layout_passes=False),
    )(tt)


def _sc_gather(tables_flat, idx2d):
    n_chunks = idx2d.shape[0]
    nrows = n_chunks * _CHUNK
    chunks_per_w = n_chunks // _NW
    groups = chunks_per_w // _GROUP

    mesh = plsc.VectorSubcoreMesh(core_axis_name="c", subcore_axis_name="s")

    def body(tables_hbm, idx_hbm, out_hbm, idx_v, rows_v, sem):
        wid = lax.axis_index("s") * _NC + lax.axis_index("c")
        crow = wid * chunks_per_w
        pltpu.sync_copy(idx_hbm.at[pl.ds(crow, chunks_per_w)], idx_v)

        def group(g, carry):
            descs = []
            for s in range(_GROUP):
                j = g * _GROUP + s
                d = pltpu.async_copy(
                    tables_hbm.at[idx_v.at[j]],
                    rows_v.at[pl.ds(s * _CHUNK, _CHUNK)],
                    sem,
                )
                descs.append(d)
            for d in descs:
                d.wait()
            pltpu.sync_copy(
                rows_v,
                out_hbm.at[pl.ds((crow + g * _GROUP) * _CHUNK, _GROUP_ROWS)],
            )
            return carry

        lax.fori_loop(0, groups, group, 0)

    return pl.kernel(
        body,
        out_type=jax.ShapeDtypeStruct((nrows, _EMB), jnp.float32),
        mesh=mesh,
        scratch_types=[
            pltpu.VMEM((chunks_per_w, _CHUNK), jnp.int32),
            pltpu.VMEM((_GROUP_ROWS, _EMB), jnp.float32),
            pltpu.SemaphoreType.DMA,
        ],
        compiler_params=pltpu.CompilerParams(use_tc_tiling_on_sc=False),
    )(tables_flat, idx2d)


def _tc_mlp(xw, emb, Ww, W1, W2, W3, b1r, g1r, be1r, b2r, g2r, be2r, c0):
    B, deep_emb = emb.shape
    R = 2048
    inv = float(1.0 / np.sqrt(1.0 + _EPS))

    def body(xw_ref, emb_ref, ww_ref, w1_ref, w2_ref, w3_ref,
             b1_ref, g1_ref, be1_ref, b2_ref, g2_ref, be2_ref, c0_ref,
             out_ref):
        xw_blk = xw_ref[...]
        emb_blk = emb_ref[...]
        dn = (((1,), (1,)), ((), ()))
        hi = jax.lax.Precision.HIGHEST
        wide = lax.dot_general(xw_blk, ww_ref[...], dn, precision=hi,
                               preferred_element_type=jnp.float32)
        w1 = w1_ref[...]
        h = lax.dot_general(xw_blk[:, :26], w1[:, :26], dn, precision=hi,
                            preferred_element_type=jnp.float32)
        h = h + lax.dot_general(emb_blk, w1[:, 26:], dn, precision=hi,
                                preferred_element_type=jnp.float32)
        h = (h + b1_ref[...]) * (g1_ref[...] * inv) + be1_ref[...]
        h = jnp.maximum(h, 0.0)
        h = lax.dot_general(h, w2_ref[...], dn, precision=hi,
                            preferred_element_type=jnp.float32)
        h = (h + b2_ref[...]) * (g2_ref[...] * inv) + be2_ref[...]
        h = jnp.maximum(h, 0.0)
        deep = lax.dot_general(h, w3_ref[...], dn, precision=hi,
                               preferred_element_type=jnp.float32)
        out_ref[...] = 0.5 * wide + 0.5 * deep + c0_ref[...]

    full = lambda shape: pl.BlockSpec(shape, lambda i: (0,) * len(shape))
    return pl.pallas_call(
        body,
        grid=(B // R,),
        in_specs=[
            pl.BlockSpec((R, xw.shape[1]), lambda i: (i, 0)),
            pl.BlockSpec((R, deep_emb), lambda i: (i, 0)),
            full(Ww.shape),
            full(W1.shape),
            full(W2.shape),
            full(W3.shape),
            full(b1r.shape),
            full(g1r.shape),
            full(be1r.shape),
            full(b2r.shape),
            full(g2r.shape),
            full(be2r.shape),
            full(c0.shape),
        ],
        out_specs=pl.BlockSpec((R, 1), lambda i: (i, 0)),
        out_shape=jax.ShapeDtypeStruct((B, 1), jnp.float32),
    )(xw, emb, Ww, W1, W2, W3, b1r, g1r, be1r, b2r, g2r, be2r, c0)


def kernel(continuous, binary, categorical, W_wide, b_wide, emb_tables,
           W1, b1, g1, be1, W2, b2, g2, be2, W3, b3):
    B = continuous.shape[0]
    catf = categorical.astype(jnp.float32)
    xw = jnp.concatenate([continuous, binary, catf], axis=1)

    cat32 = categorical.astype(jnp.int32)
    offs = jnp.arange(_N_CAT, dtype=jnp.int32) * _VOCAB
    idx2d = (cat32 + offs[None, :]).reshape(-1, _CHUNK)

    tt = jnp.transpose(emb_tables, (0, 2, 1))
    tables_flat = _sc_detile(tt).reshape(_N_CAT * _VOCAB, _EMB)
    del tt
    gathered = _sc_gather(tables_flat, idx2d)
    emb_flat = gathered.reshape(B, _N_CAT * _EMB)

    c0 = (0.5 * (b_wide + b3)).reshape(1, 1)
    out2d = _tc_mlp(
        xw, emb_flat, W_wide, W1, W2, W3,
        b1.reshape(1, -1), g1.reshape(1, -1), be1.reshape(1, -1),
        b2.reshape(1, -1), g2.reshape(1, -1), be2.reshape(1, -1),
        c0,
    )
    return out2d.reshape(B)
